# baseline (speedup 1.0000x reference)
import functools

import jax
import jax.numpy as jnp
from jax import lax
from jax.experimental import pallas as pl
from jax.experimental.pallas import tpu as pltpu
from jax.experimental.pallas import tpu_sc as plsc

_CODEBOOK = 1024
_D = 256
_ROWS_PER_BLOCK = 4096

_NC = 2
_NS = 16
_NW = _NC * _NS


def _vq_block(z_ref, emb_ref, idx_ref, acc_ref, esq_ref, m2e_ref):
    i = pl.program_id(0)
    zb = z_ref[...]

    @pl.when(i == 0)
    def _prep_codebook():
        emb = emb_ref[...]
        ones = jnp.ones((_D, 1), jnp.float32)
        esq_ref[...] = jax.lax.dot_general(
            emb * emb, ones, (((1,), (0,)), ((), ())),
            preferred_element_type=jnp.float32)
        m2e_ref[...] = emb * (-2.0)

    e_sq = esq_ref[...]
    ones_row = jnp.ones((1, _D), jnp.float32)
    z_sq = jax.lax.dot_general(
        ones_row, zb * zb, (((1,), (1,)), ((), ())),
        preferred_element_type=jnp.float32)
    scores2 = jax.lax.dot_general(
        m2e_ref[...], zb, (((1,), (1,)), ((), ())),
        preferred_element_type=jnp.float32)
    d = (z_sq + e_sq) + scores2
    min_d = jnp.min(d, axis=0, keepdims=True)
    row = jax.lax.broadcasted_iota(jnp.int32, d.shape, 0)
    idx = jnp.min(jnp.where(d == min_d, row, jnp.int32(_CODEBOOK)),
                  axis=0, keepdims=True)
    idx_ref[...] = idx.reshape(1, 1, idx.shape[-1])
    part = jnp.sum(min_d).reshape(1, 1)

    @pl.when(i == 0)
    def _init():
        acc_ref[...] = part

    @pl.when(i != 0)
    def _accum():
        acc_ref[...] += part


def _distance_argmin(z_flat, embedding, n_rows, row_offset):
    d_ = z_flat.shape[1]
    nb = n_rows // _ROWS_PER_BLOCK
    boff = row_offset // _ROWS_PER_BLOCK
    return pl.pallas_call(
        _vq_block,
        grid=(nb,),
        in_specs=[
            pl.BlockSpec((_ROWS_PER_BLOCK, d_), lambda i: (i + boff, 0)),
            pl.BlockSpec((_CODEBOOK, d_), lambda i: (0, 0)),
        ],
        out_specs=[
            pl.BlockSpec((1, 1, _ROWS_PER_BLOCK), lambda i: (i, 0, 0)),
            pl.BlockSpec((1, 1), lambda i: (0, 0)),
        ],
        out_shape=[
            jax.ShapeDtypeStruct((nb, 1, _ROWS_PER_BLOCK), jnp.int32),
            jax.ShapeDtypeStruct((1, 1), jnp.float32),
        ],
        scratch_shapes=[
            pltpu.VMEM((_CODEBOOK, 1), jnp.float32),
            pltpu.VMEM((_CODEBOOK, _D), jnp.float32),
        ],
    )(z_flat, embedding)


def _make_gather(n_rows):
    rows_per_w = n_rows // _NW
    chunk = 128
    n_chunks = rows_per_w // chunk
    mesh = plsc.VectorSubcoreMesh(core_axis_name="c", subcore_axis_name="s")

    @functools.partial(
        pl.kernel, mesh=mesh,
        out_type=jax.ShapeDtypeStruct((n_rows, _D), jnp.float32),
        scratch_types=[
            pltpu.VMEM((rows_per_w,), jnp.int32),
            pltpu.VMEM((chunk, _D), jnp.float32),
            pltpu.VMEM((chunk, _D), jnp.float32),
            pltpu.SemaphoreType.DMA,
            pltpu.SemaphoreType.DMA,
            pltpu.SemaphoreType.DMA,
            pltpu.SemaphoreType.DMA,
        ],
    )
    def gather(table_hbm, idx_hbm, out_hbm, idx_v, rows_a, rows_b,
               gsem_a, gsem_b, wsem_a, wsem_b):
        wid = lax.axis_index("s") * _NC + lax.axis_index("c")
        base = wid * rows_per_w
        pltpu.sync_copy(idx_hbm.at[pl.ds(base, rows_per_w)], idx_v)
        bufs = (rows_a, rows_b)
        gsems = (gsem_a, gsem_b)
        wsems = (wsem_a, wsem_b)
        gathers = []
        writes = [None, None]
        for k in range(n_chunks):
            b = k % 2
            if k >= 2:
                writes[b].wait()
            gathers.append(pltpu.async_copy(
                table_hbm.at[idx_v.at[pl.ds(k * chunk, chunk)]],
                bufs[b], gsems[b]))
            if k >= 1:
                gathers[k - 1].wait()
                writes[(k - 1) % 2] = pltpu.async_copy(
                    bufs[(k - 1) % 2],
                    out_hbm.at[pl.ds(base + (k - 1) * chunk, chunk)],
                    wsems[(k - 1) % 2])
        gathers[n_chunks - 1].wait()
        writes[(n_chunks - 1) % 2] = pltpu.async_copy(
            bufs[(n_chunks - 1) % 2],
            out_hbm.at[pl.ds(base + (n_chunks - 1) * chunk, chunk)],
            wsems[(n_chunks - 1) % 2])
        writes[0].wait()
        if n_chunks > 1:
            writes[1].wait()

    return gather


@functools.partial(jax.jit, static_argnames=())
def kernel(z, embedding):
    z = z.astype(jnp.float32)
    B, T, D = z.shape
    N = B * T
    z_flat = z.reshape(N, D)

    half = N // 2
    gather_half = _make_gather(half)
    idx3_a, acc_a = _distance_argmin(z_flat, embedding, half, 0)
    zq_a = gather_half(embedding, idx3_a.reshape(half))
    idx3_b, acc_b = _distance_argmin(z_flat, embedding, half, half)
    zq_b = gather_half(embedding, idx3_b.reshape(half))

    zq = jnp.concatenate([zq_a, zq_b], axis=0)
    z_quantized = zq.reshape(B, T, D)
    indices = jnp.concatenate(
        [idx3_a.reshape(half), idx3_b.reshape(half)]).reshape(B, T)
    m = (acc_a[0, 0] + acc_b[0, 0]) / jnp.float32(N * D)
    commitment_loss = jnp.float32(0.25) * m
    codebook_loss = m
    loss = commitment_loss + codebook_loss
    return (z_quantized, loss, commitment_loss, codebook_loss, indices)


# speedup vs baseline: 1.2444x; 1.2444x over previous
import functools

import jax
import jax.numpy as jnp
from jax import lax
from jax.experimental import pallas as pl
from jax.experimental.pallas import tpu as pltpu
from jax.experimental.pallas import tpu_sc as plsc

_CODEBOOK = 1024
_D = 256
_ROWS_PER_BLOCK = 4096

_NC = 2
_NS = 16
_NW = _NC * _NS


def _vq_block(z_ref, emb_ref, idx_ref, acc_ref, esq_ref, m2e_ref):
    i = pl.program_id(0)
    zb = z_ref[...]

    @pl.when(i == 0)
    def _prep_codebook():
        emb = emb_ref[...]
        ones = jnp.ones((_D, 1), jnp.float32)
        esq_ref[...] = jax.lax.dot_general(
            emb * emb, ones, (((1,), (0,)), ((), ())),
            preferred_element_type=jnp.float32)
        m2e_ref[...] = emb * (-2.0)

    e_sq = esq_ref[...]
    ones_row = jnp.ones((1, _D), jnp.float32)
    z_sq = jax.lax.dot_general(
        ones_row, zb * zb, (((1,), (1,)), ((), ())),
        preferred_element_type=jnp.float32)
    scores2 = jax.lax.dot_general(
        m2e_ref[...], zb, (((1,), (1,)), ((), ())),
        preferred_element_type=jnp.float32)
    d = (z_sq + e_sq) + scores2
    min_d = jnp.min(d, axis=0, keepdims=True)
    idx = jnp.argmin(d, axis=0).astype(jnp.int32)
    idx_ref[...] = idx.reshape(1, 1, idx.shape[-1])
    part = jnp.sum(min_d).reshape(1, 1)

    @pl.when(i == 0)
    def _init():
        acc_ref[...] = part

    @pl.when(i != 0)
    def _accum():
        acc_ref[...] += part


def _distance_argmin(z_flat, embedding, n_rows, row_offset):
    d_ = z_flat.shape[1]
    nb = n_rows // _ROWS_PER_BLOCK
    boff = row_offset // _ROWS_PER_BLOCK
    return pl.pallas_call(
        _vq_block,
        grid=(nb,),
        in_specs=[
            pl.BlockSpec((_ROWS_PER_BLOCK, d_), lambda i: (i + boff, 0)),
            pl.BlockSpec((_CODEBOOK, d_), lambda i: (0, 0)),
        ],
        out_specs=[
            pl.BlockSpec((1, 1, _ROWS_PER_BLOCK), lambda i: (i, 0, 0)),
            pl.BlockSpec((1, 1), lambda i: (0, 0)),
        ],
        out_shape=[
            jax.ShapeDtypeStruct((nb, 1, _ROWS_PER_BLOCK), jnp.int32),
            jax.ShapeDtypeStruct((1, 1), jnp.float32),
        ],
        scratch_shapes=[
            pltpu.VMEM((_CODEBOOK, 1), jnp.float32),
            pltpu.VMEM((_CODEBOOK, _D), jnp.float32),
        ],
    )(z_flat, embedding)


def _make_gather(n_rows):
    rows_per_w = n_rows // _NW
    chunk = 128
    n_chunks = rows_per_w // chunk
    mesh = plsc.VectorSubcoreMesh(core_axis_name="c", subcore_axis_name="s")

    @functools.partial(
        pl.kernel, mesh=mesh,
        out_type=jax.ShapeDtypeStruct((n_rows, _D), jnp.float32),
        scratch_types=[
            pltpu.VMEM((rows_per_w,), jnp.int32),
            pltpu.VMEM((chunk, _D), jnp.float32),
            pltpu.VMEM((chunk, _D), jnp.float32),
            pltpu.SemaphoreType.DMA,
            pltpu.SemaphoreType.DMA,
            pltpu.SemaphoreType.DMA,
            pltpu.SemaphoreType.DMA,
        ],
    )
    def gather(table_hbm, idx_hbm, out_hbm, idx_v, rows_a, rows_b,
               gsem_a, gsem_b, wsem_a, wsem_b):
        wid = lax.axis_index("s") * _NC + lax.axis_index("c")
        base = wid * rows_per_w
        pltpu.sync_copy(idx_hbm.at[pl.ds(base, rows_per_w)], idx_v)
        bufs = (rows_a, rows_b)
        gsems = (gsem_a, gsem_b)
        wsems = (wsem_a, wsem_b)
        gathers = []
        writes = [None, None]
        for k in range(n_chunks):
            b = k % 2
            if k >= 2:
                writes[b].wait()
            gathers.append(pltpu.async_copy(
                table_hbm.at[idx_v.at[pl.ds(k * chunk, chunk)]],
                bufs[b], gsems[b]))
            if k >= 1:
                gathers[k - 1].wait()
                writes[(k - 1) % 2] = pltpu.async_copy(
                    bufs[(k - 1) % 2],
                    out_hbm.at[pl.ds(base + (k - 1) * chunk, chunk)],
                    wsems[(k - 1) % 2])
        gathers[n_chunks - 1].wait()
        writes[(n_chunks - 1) % 2] = pltpu.async_copy(
            bufs[(n_chunks - 1) % 2],
            out_hbm.at[pl.ds(base + (n_chunks - 1) * chunk, chunk)],
            wsems[(n_chunks - 1) % 2])
        writes[0].wait()
        if n_chunks > 1:
            writes[1].wait()

    return gather


@functools.partial(jax.jit, static_argnames=())
def kernel(z, embedding):
    z = z.astype(jnp.float32)
    B, T, D = z.shape
    N = B * T
    z_flat = z.reshape(N, D)

    idx3, acc = _distance_argmin(z_flat, embedding, N, 0)
    zq = _make_gather(N)(embedding, idx3.reshape(N))

    z_quantized = zq.reshape(B, T, D)
    indices = idx3.reshape(B, T)
    m = acc[0, 0] / jnp.float32(N * D)
    commitment_loss = jnp.float32(0.25) * m
    codebook_loss = m
    loss = commitment_loss + codebook_loss
    return (z_quantized, loss, commitment_loss, codebook_loss, indices)
